# baseline (speedup 1.0000x reference)
import functools

import jax
import jax.numpy as jnp
from jax import lax
from jax.experimental import pallas as pl
from jax.experimental.pallas import tpu as pltpu
from jax.experimental.pallas import tpu_sc as plsc

DEPTH = 1000
ROWS = 1024 * 26
NC, NS = 2, 16
NW = NC * NS
ROWS_PER_W = ROWS // NW
BLK = 16
WORDS = BLK * DEPTH
NBUF = 4
ITERS = ROWS_PER_W // BLK


def _one_hot_sc(x_flat, zero_blk):
    mesh = plsc.VectorSubcoreMesh(core_axis_name="c", subcore_axis_name="s")

    @functools.partial(
        pl.kernel,
        mesh=mesh,
        out_type=jax.ShapeDtypeStruct((ROWS * DEPTH,), jnp.int32),
        compiler_params=pltpu.CompilerParams(needs_layout_passes=False),
        scratch_types=(
            [pltpu.VMEM((ROWS_PER_W,), jnp.int32)]
            + [pltpu.VMEM((NBUF * 16,), jnp.int32)]
            + [pltpu.VMEM((WORDS,), jnp.int32) for _ in range(NBUF)]
            + [pltpu.SemaphoreType.DMA for _ in range(NBUF)]
        ),
    )
    def k(x_hbm, z_hbm, out_hbm, idx_v, offs_v, *rest):
        bufs = rest[:NBUF]
        sems = rest[NBUF:]
        wid = lax.axis_index("s") * NC + lax.axis_index("c")
        base_row = wid * ROWS_PER_W

        pltpu.sync_copy(x_hbm.at[pl.ds(base_row * 1, ROWS_PER_W)], idx_v)
        for b in range(NBUF):
            pltpu.sync_copy(z_hbm, bufs[b])

        lane_base = lax.iota(jnp.int32, 16) * DEPTH
        ones = jnp.full((16,), 1, jnp.int32)
        zeros = jnp.zeros((16,), jnp.int32)

        def poke_and_fire(i, b):
            offs = idx_v[pl.ds(i * BLK, BLK)] + lane_base
            plsc.store_scatter(bufs[b], [offs], ones)
            offs_v[pl.ds(b * 16, 16)] = offs
            dst = out_hbm.at[pl.ds((base_row + i * BLK) * DEPTH, WORDS)]
            pltpu.async_copy(bufs[b], dst, sems[b])

        for b in range(NBUF):
            poke_and_fire(b, b)

        def body(step, carry):
            for b in range(NBUF):
                i = NBUF + step * NBUF + b
                dst = out_hbm.at[pl.ds((base_row + i * BLK) * DEPTH, WORDS)]
                pltpu.make_async_copy(bufs[b], dst, sems[b]).wait()
                prev = offs_v[pl.ds(b * 16, 16)]
                plsc.store_scatter(bufs[b], [prev], zeros)
                poke_and_fire(i, b)
            return carry

        lax.fori_loop(0, (ITERS - NBUF) // NBUF, body, 0)

        for b in range(NBUF):
            dst = out_hbm.at[pl.ds(base_row * DEPTH, WORDS)]
            pltpu.make_async_copy(bufs[b], dst, sems[b]).wait()

    return k(x_flat, zero_blk)


def kernel(x):
    x_flat = x.reshape(-1)
    zero_blk = jnp.zeros((WORDS,), jnp.int32)
    out = _one_hot_sc(x_flat, zero_blk)
    return out.reshape(x.shape + (DEPTH,))


# speedup vs baseline: 1.0187x; 1.0187x over previous
import functools

import jax
import jax.numpy as jnp
from jax import lax
from jax.experimental import pallas as pl
from jax.experimental.pallas import tpu as pltpu
from jax.experimental.pallas import tpu_sc as plsc

DEPTH = 1000
ROWS = 1024 * 26
NC, NS = 2, 16
NW = NC * NS
ROWS_PER_W = ROWS // NW
BLK = 32
NSUB = BLK // 16
WORDS = BLK * DEPTH
NBUF = 2
ITERS = ROWS_PER_W // BLK


def _one_hot_sc(x_flat, zero_blk):
    mesh = plsc.VectorSubcoreMesh(core_axis_name="c", subcore_axis_name="s")

    @functools.partial(
        pl.kernel,
        mesh=mesh,
        out_type=jax.ShapeDtypeStruct((ROWS * DEPTH,), jnp.int32),
        compiler_params=pltpu.CompilerParams(needs_layout_passes=False),
        scratch_types=(
            [pltpu.VMEM((ROWS_PER_W,), jnp.int32)]
            + [pltpu.VMEM((NBUF * BLK,), jnp.int32)]
            + [pltpu.VMEM((WORDS,), jnp.int32) for _ in range(NBUF)]
            + [pltpu.SemaphoreType.DMA for _ in range(NBUF)]
        ),
    )
    def k(x_hbm, z_hbm, out_hbm, idx_v, offs_v, *rest):
        bufs = rest[:NBUF]
        sems = rest[NBUF:]
        wid = lax.axis_index("s") * NC + lax.axis_index("c")
        base_row = wid * ROWS_PER_W

        pltpu.sync_copy(x_hbm.at[pl.ds(base_row * 1, ROWS_PER_W)], idx_v)
        for b in range(NBUF):
            pltpu.sync_copy(z_hbm, bufs[b])

        lane_base = lax.iota(jnp.int32, 16) * DEPTH
        ones = jnp.full((16,), 1, jnp.int32)
        zeros = jnp.zeros((16,), jnp.int32)

        def poke_and_fire(i, b):
            for s in range(NSUB):
                offs = idx_v[pl.ds(i * BLK + s * 16, 16)] + lane_base
                plsc.store_scatter(bufs[b], [offs + s * 16 * DEPTH], ones)
                offs_v[pl.ds(b * BLK + s * 16, 16)] = offs + s * 16 * DEPTH
            dst = out_hbm.at[pl.ds((base_row + i * BLK) * DEPTH, WORDS)]
            pltpu.async_copy(bufs[b], dst, sems[b])

        for b in range(NBUF):
            poke_and_fire(b, b)

        def body(step, carry):
            for b in range(NBUF):
                i = NBUF + step * NBUF + b
                dst = out_hbm.at[pl.ds((base_row + i * BLK) * DEPTH, WORDS)]
                pltpu.make_async_copy(bufs[b], dst, sems[b]).wait()
                for s in range(NSUB):
                    prev = offs_v[pl.ds(b * BLK + s * 16, 16)]
                    plsc.store_scatter(bufs[b], [prev], zeros)
                poke_and_fire(i, b)
            return carry

        lax.fori_loop(0, (ITERS - NBUF) // NBUF, body, 0)

        for b in range(NBUF):
            dst = out_hbm.at[pl.ds(base_row * DEPTH, WORDS)]
            pltpu.make_async_copy(bufs[b], dst, sems[b]).wait()

    return k(x_flat, zero_blk)


def kernel(x):
    x_flat = x.reshape(-1)
    zero_blk = jnp.zeros((WORDS,), jnp.int32)
    out = _one_hot_sc(x_flat, zero_blk)
    return out.reshape(x.shape + (DEPTH,))


# speedup vs baseline: 1.4252x; 1.3990x over previous
import functools

import jax
import jax.numpy as jnp
from jax import lax
from jax.experimental import pallas as pl
from jax.experimental.pallas import tpu as pltpu
from jax.experimental.pallas import tpu_sc as plsc

DEPTH = 1000
ROWS = 1024 * 26
NC, NS = 2, 16
NW = NC * NS
ROWS_PER_W = ROWS // NW
BLK = 32
NSUB = BLK // 16
NBUF = 2
ITERS = ROWS_PER_W // BLK


def _one_hot_sc(x_flat, zero_blk):
    mesh = plsc.VectorSubcoreMesh(core_axis_name="c", subcore_axis_name="s")

    @functools.partial(
        pl.kernel,
        mesh=mesh,
        out_type=jax.ShapeDtypeStruct((ROWS, DEPTH), jnp.int32),
        compiler_params=pltpu.CompilerParams(needs_layout_passes=False),
        scratch_types=(
            [pltpu.VMEM((ROWS_PER_W,), jnp.int32)]
            + [pltpu.VMEM((NBUF * BLK,), jnp.int32)]
            + [pltpu.VMEM((BLK, DEPTH), jnp.int32) for _ in range(NBUF)]
            + [pltpu.SemaphoreType.DMA for _ in range(NBUF)]
        ),
    )
    def k(x_hbm, z_hbm, out_hbm, idx_v, offs_v, *rest):
        bufs = rest[:NBUF]
        sems = rest[NBUF:]
        wid = lax.axis_index("s") * NC + lax.axis_index("c")
        base_row = wid * ROWS_PER_W

        pltpu.sync_copy(x_hbm.at[pl.ds(base_row, ROWS_PER_W)], idx_v)
        for b in range(NBUF):
            pltpu.sync_copy(z_hbm, bufs[b])

        lane = lax.iota(jnp.int32, 16)
        ones = jnp.full((16,), 1, jnp.int32)
        zeros = jnp.zeros((16,), jnp.int32)

        def poke_and_fire(i, b):
            for s in range(NSUB):
                cols = idx_v[pl.ds(i * BLK + s * 16, 16)]
                plsc.store_scatter(bufs[b], [lane + s * 16, cols], ones)
                offs_v[pl.ds(b * BLK + s * 16, 16)] = cols
            dst = out_hbm.at[pl.ds(base_row + i * BLK, BLK)]
            pltpu.async_copy(bufs[b], dst, sems[b])

        for b in range(NBUF):
            poke_and_fire(b, b)

        def body(step, carry):
            for b in range(NBUF):
                i = NBUF + step * NBUF + b
                dst = out_hbm.at[pl.ds(base_row + i * BLK, BLK)]
                pltpu.make_async_copy(bufs[b], dst, sems[b]).wait()
                for s in range(NSUB):
                    prev = offs_v[pl.ds(b * BLK + s * 16, 16)]
                    plsc.store_scatter(bufs[b], [lane + s * 16, prev], zeros)
                poke_and_fire(i, b)
            return carry

        lax.fori_loop(0, (ITERS - NBUF) // NBUF, body, 0)

        for b in range(NBUF):
            dst = out_hbm.at[pl.ds(base_row, BLK)]
            pltpu.make_async_copy(bufs[b], dst, sems[b]).wait()

    return k(x_flat, zero_blk)


def kernel(x):
    x_flat = x.reshape(-1)
    zero_blk = jnp.zeros((BLK, DEPTH), jnp.int32)
    out = _one_hot_sc(x_flat, zero_blk)
    return out.reshape(x.shape + (DEPTH,))


# speedup vs baseline: 1.8006x; 1.2634x over previous
import functools

import jax
import jax.numpy as jnp
from jax import lax
from jax.experimental import pallas as pl
from jax.experimental.pallas import tpu as pltpu
from jax.experimental.pallas import tpu_sc as plsc

DEPTH = 1000
B0, B1 = 1024, 26
NC, NS = 2, 16
NW = NC * NS
PL_PER_W = B0 // NW
VALS_PER_W = PL_PER_W * B1
NBUF = 2
IDX_PAD = 16


def _one_hot_sc(x_flat, zero_plane):
    mesh = plsc.VectorSubcoreMesh(core_axis_name="c", subcore_axis_name="s")

    @functools.partial(
        pl.kernel,
        mesh=mesh,
        out_type=jax.ShapeDtypeStruct((B0, B1, DEPTH), jnp.int32),
        compiler_params=pltpu.CompilerParams(needs_layout_passes=False),
        scratch_types=(
            [pltpu.VMEM((VALS_PER_W + IDX_PAD,), jnp.int32)]
            + [pltpu.VMEM((NBUF * 32,), jnp.int32)]
            + [pltpu.VMEM((1, B1, DEPTH), jnp.int32) for _ in range(NBUF)]
            + [pltpu.SemaphoreType.DMA for _ in range(NBUF)]
        ),
    )
    def k(x_hbm, z_hbm, out_hbm, idx_v, cols_v, *rest):
        bufs = rest[:NBUF]
        sems = rest[NBUF:]
        wid = lax.axis_index("s") * NC + lax.axis_index("c")
        base_pl = wid * PL_PER_W

        pltpu.sync_copy(x_hbm.at[pl.ds(wid * VALS_PER_W, VALS_PER_W)],
                        idx_v.at[pl.ds(0, VALS_PER_W)])
        for b in range(NBUF):
            pltpu.sync_copy(z_hbm, bufs[b])

        lane = lax.iota(jnp.int32, 16)
        plane0 = jnp.zeros((16,), jnp.int32)
        ones = jnp.full((16,), 1, jnp.int32)
        zeros = jnp.zeros((16,), jnp.int32)
        tail_mask = lane < (B1 - 16)

        def scatter_plane(b, i, vals, use_saved_cols):
            for s, mask in ((0, None), (1, tail_mask)):
                slot = b * 32 + s * 16
                if use_saved_cols:
                    cols = cols_v[pl.ds(slot, 16)]
                else:
                    cols = idx_v[pl.ds(i * B1 + s * 16, 16)]
                    cols_v[pl.ds(slot, 16)] = cols
                plsc.store_scatter(bufs[b], [plane0, lane + s * 16, cols],
                                   vals, mask=mask)

        def poke_and_fire(i, b):
            scatter_plane(b, i, ones, use_saved_cols=False)
            dst = out_hbm.at[pl.ds(base_pl + i, 1)]
            pltpu.async_copy(bufs[b], dst, sems[b])

        for b in range(NBUF):
            poke_and_fire(b, b)

        def body(step, carry):
            for b in range(NBUF):
                i = NBUF + step * NBUF + b
                dst = out_hbm.at[pl.ds(base_pl + i, 1)]
                pltpu.make_async_copy(bufs[b], dst, sems[b]).wait()
                scatter_plane(b, i, zeros, use_saved_cols=True)
                poke_and_fire(i, b)
            return carry

        lax.fori_loop(0, (PL_PER_W - NBUF) // NBUF, body, 0)

        for b in range(NBUF):
            dst = out_hbm.at[pl.ds(base_pl, 1)]
            pltpu.make_async_copy(bufs[b], dst, sems[b]).wait()

    return k(x_flat, zero_plane)


def kernel(x):
    x_flat = x.reshape(-1)
    zero_plane = jnp.zeros((1, B1, DEPTH), jnp.int32)
    return _one_hot_sc(x_flat, zero_plane)


# speedup vs baseline: 4.9100x; 2.7269x over previous
import functools

import jax
import jax.numpy as jnp
from jax import lax
from jax.experimental import pallas as pl
from jax.experimental.pallas import tpu as pltpu
from jax.experimental.pallas import tpu_sc as plsc

DEPTH = 1000
B0, B1 = 1024, 26
NC, NS = 2, 16
NW = NC * NS
COLS = 128
NCHUNK = B1 * (B0 // COLS)
NT = -(-NCHUNK // NW)


def _one_hot_sc(xt_flat, zero_chunk):
    mesh = plsc.VectorSubcoreMesh(core_axis_name="c", subcore_axis_name="s")

    @functools.partial(
        pl.kernel,
        mesh=mesh,
        out_type=jax.ShapeDtypeStruct((B1, DEPTH, B0), jnp.int32),
        compiler_params=pltpu.CompilerParams(needs_layout_passes=False),
        scratch_types=[
            pltpu.VMEM((COLS,), jnp.int32),
            pltpu.VMEM((DEPTH, COLS), jnp.int32),
            pltpu.SemaphoreType.DMA,
        ],
    )
    def k(xt_hbm, z_hbm, out_hbm, xv, buf, sem):
        wid = lax.axis_index("s") * NC + lax.axis_index("c")

        pltpu.sync_copy(z_hbm, buf)

        lane = lax.iota(jnp.int32, 16)
        ones = jnp.full((16,), 1, jnp.int32)
        zeros = jnp.zeros((16,), jnp.int32)

        def scatter_chunk(vals):
            for s in range(COLS // 16):
                rows = xv[pl.ds(s * 16, 16)]
                plsc.store_scatter(buf, [rows, lane + s * 16], vals)

        for t in range(NT):
            g = t * NW + wid
            j = g // (B0 // COLS)
            c = g % (B0 // COLS)

            @pl.when(g < NCHUNK)
            def _():
                if t > 0:
                    pltpu.make_async_copy(
                        buf, out_hbm.at[0, :, pl.ds(0, COLS)], sem).wait()
                    scatter_chunk(zeros)
                pltpu.sync_copy(
                    xt_hbm.at[pl.ds(j * B0 + c * COLS, COLS)], xv)
                scatter_chunk(ones)
                pltpu.async_copy(
                    buf, out_hbm.at[j, :, pl.ds(c * COLS, COLS)], sem)

        pltpu.make_async_copy(buf, out_hbm.at[0, :, pl.ds(0, COLS)],
                              sem).wait()

    return k(xt_flat, zero_chunk)


def kernel(x):
    xt_flat = x.T.reshape(-1)
    zero_chunk = jnp.zeros((DEPTH, COLS), jnp.int32)
    out_t = _one_hot_sc(xt_flat, zero_chunk)
    return jnp.transpose(out_t, (2, 0, 1))
